# Initial kernel scaffold; baseline (speedup 1.0000x reference)
import jax
import jax.numpy as jnp
from jax.experimental import pallas as pl


def kernel(s, out_nbrs, E_bias_flat, col_id, W_c1, b_c1, W_c2, b_c2, W_q1, b_q1, W_q2, b_q2, W_k1, b_k1, W_k2, b_k2, W_u1, b_u1, W_u2, b_u2):
    raise NotImplementedError("write your pallas kernel here")


# speedup vs baseline: 3.6995x; 3.6995x over previous
import functools

import jax
import jax.numpy as jnp
from jax import lax
from jax.experimental import pallas as pl
from jax.experimental.pallas import tpu as pltpu
from jax.experimental.pallas import tpu_sc as plsc

N = 10000
K = 32
D_s = 128
D_id = 32
HD = 64
FF = 256

NC = 2
NS = 16
NW = NC * NS

N_PAD = 10240
E_PAD = N_PAD * K
EPW = E_PAD // NW
CH = 128
CHUNKS = EPW // CH
IDXROWS = E_PAD // CH

BN1 = 512
BN2 = 256

@functools.cache
def _sc_mesh():
    return plsc.VectorSubcoreMesh(core_axis_name="c", subcore_axis_name="s",
                                  num_cores=NC, num_subcores=NS)


def _rms(x):
    return x * lax.rsqrt(jnp.mean(x * x, axis=-1, keepdims=True) + 1e-6)


def _tc1_body(s_ref, cid_ref, wc1a, wc1b, bc1, wc2, bc2,
              wq1a, wq1b, bq1, wq2, bq2, wk1, bk1, wk2, bk2,
              m_ref, q_ref, k_ref):
    x = s_ref[...]
    cid = cid_ref[...]
    sn = _rms(x)
    h = jax.nn.gelu(sn @ wc1a[...] + cid @ wc1b[...] + bc1[...])
    m_ref[...] = h @ wc2[...] + bc2[...]
    hq = jax.nn.gelu(x @ wq1a[...] + cid @ wq1b[...] + bq1[...])
    q_ref[...] = hq @ wq2[...] + bq2[...]
    hk = jax.nn.gelu(cid @ wk1[...] + bk1[...])
    k_ref[...] = hk @ wk2[...] + bk2[...]


def _tc1(s_pad, cid_pad, wc1a, wc1b, bc1, wc2, bc2,
         wq1a, wq1b, bq1, wq2, bq2, wk1, bk1, wk2, bk2):
    g = N_PAD // BN1
    row = lambda i: (i, 0)
    full = lambda i: (0, 0)
    wspec = lambda a: pl.BlockSpec(a.shape, full)
    return pl.pallas_call(
        _tc1_body,
        grid=(g,),
        in_specs=[pl.BlockSpec((BN1, D_s), row), pl.BlockSpec((BN1, D_id), row)]
        + [wspec(a) for a in (wc1a, wc1b, bc1, wc2, bc2,
                              wq1a, wq1b, bq1, wq2, bq2, wk1, bk1, wk2, bk2)],
        out_specs=[pl.BlockSpec((BN1, D_s), row),
                   pl.BlockSpec((BN1, HD), row),
                   pl.BlockSpec((BN1, HD), row)],
        out_shape=[jax.ShapeDtypeStruct((N_PAD, D_s), jnp.float32),
                   jax.ShapeDtypeStruct((N_PAD, HD), jnp.float32),
                   jax.ShapeDtypeStruct((N_PAD, HD), jnp.float32)],
    )(s_pad, cid_pad, wc1a, wc1b, bc1, wc2, bc2,
      wq1a, wq1b, bq1, wq2, bq2, wk1, bk1, wk2, bk2)


def _sc_gather_body(k_hbm, idx_hbm, out_hbm, idx_v, rows_v, sem):
    wid = lax.axis_index("s") * NC + lax.axis_index("c")
    base_idx_row = wid * CHUNKS
    pltpu.sync_copy(idx_hbm.at[pl.ds(base_idx_row, CHUNKS)], idx_v)

    G = 4

    def step(i, _):
        descs = []
        for g in range(G):
            j = i * G + g
            d = pltpu.async_copy(k_hbm.at[idx_v.at[j]],
                                 rows_v.at[pl.ds(g * CH, CH)], sem)
            descs.append(d)
        for d in descs:
            d.wait()
        pltpu.sync_copy(
            rows_v, out_hbm.at[pl.ds(wid * EPW + i * (G * CH), G * CH)])
        return 0

    lax.fori_loop(0, CHUNKS // G, step, 0)


@functools.cache
def _sc_gather_kernel():
    return pl.kernel(
        _sc_gather_body,
        out_type=jax.ShapeDtypeStruct((E_PAD, HD), jnp.float32),
        mesh=_sc_mesh(),
        scratch_types=[pltpu.VMEM((CHUNKS, CH), jnp.int32),
                       pltpu.VMEM((4 * CH, HD), jnp.float32),
                       pltpu.SemaphoreType.DMA],
        compiler_params=pltpu.CompilerParams(use_tc_tiling_on_sc=False),
    )


def _sc_gather(kvec, idx_flat):
    return _sc_gather_kernel()(kvec, idx_flat)


def _tc3_body(q_ref, knbr_ref, eb_ref, m_ref, msgs_ref):
    i = pl.program_id(0)
    qb = q_ref[...]
    kn = knbr_ref[...].reshape(BN2, K, HD)
    sc = jnp.sum(qb[:, None, :] * kn, axis=-1) + eb_ref[...]
    row = i * BN2 + lax.broadcasted_iota(jnp.int32, (BN2, 1), 0)
    w = jnp.where(row < N, jax.nn.sigmoid(sc), 0.0)
    msgs_ref[...] = (w[:, :, None] * m_ref[...][:, None, :]).reshape(BN2 * K, D_s)


def _tc3(q, knbr, eb_pad, m):
    g = N_PAD // BN2
    row = lambda i: (i, 0)
    return pl.pallas_call(
        _tc3_body,
        grid=(g,),
        in_specs=[pl.BlockSpec((BN2, HD), row),
                  pl.BlockSpec((BN2 * K, HD), row),
                  pl.BlockSpec((BN2, K), row),
                  pl.BlockSpec((BN2, D_s), row)],
        out_specs=pl.BlockSpec((BN2 * K, D_s), row),
        out_shape=jax.ShapeDtypeStruct((E_PAD, D_s), jnp.float32),
    )(q, knbr, eb_pad, m)


def _sc_scatter_body(msgs_hbm, dst_hbm, zer_hbm, out_hbm,
                     dst_v, msgs_v, sem, acc):
    c = lax.axis_index("c")
    sid = lax.axis_index("s")
    wid = sid * NC + c
    rows_per_tile = N_PAD // NS
    zbase = sid * rows_per_tile

    pltpu.sync_copy(dst_hbm.at[pl.ds(wid * CHUNKS, CHUNKS)], dst_v)
    for j in range(rows_per_tile // CH):
        pltpu.sync_copy(zer_hbm, acc.at[pl.ds(zbase + j * CH, CH)])
    plsc.subcore_barrier()

    def step(j, _):
        pltpu.sync_copy(msgs_hbm.at[pl.ds(wid * EPW + j * CH, CH)], msgs_v)
        pltpu.sync_copy(msgs_v, acc.at[dst_v.at[j]], add=True)
        return 0

    lax.fori_loop(0, CHUNKS, step, 0)
    plsc.subcore_barrier()

    for j in range(rows_per_tile // CH):
        r = zbase + j * CH
        pltpu.sync_copy(acc.at[pl.ds(r, CH)],
                        out_hbm.at[pl.ds(c * N_PAD + r, CH)])


@functools.cache
def _sc_scatter_kernel():
    return pl.kernel(
        _sc_scatter_body,
        out_type=jax.ShapeDtypeStruct((NC * N_PAD, D_s), jnp.float32),
        mesh=_sc_mesh(),
        scratch_types=[pltpu.VMEM((CHUNKS, CH), jnp.int32),
                       pltpu.VMEM((CH, D_s), jnp.float32),
                       pltpu.SemaphoreType.DMA,
                       pltpu.VMEM_SHARED((N_PAD, D_s), jnp.float32)],
    )


def _sc_scatter(msgs, idx_flat, zer):
    return _sc_scatter_kernel()(msgs, idx_flat, zer)


def _tc5_body(s_ref, cid_ref, inc0_ref, inc1_ref,
              wu1a, wu1b, wu1c, bu1, wu2, bu2, out_ref):
    x = s_ref[...]
    cid = cid_ref[...]
    inc = inc0_ref[...] + inc1_ref[...]
    h = jax.nn.gelu(x @ wu1a[...] + cid @ wu1b[...] + inc @ wu1c[...] + bu1[...])
    out_ref[...] = x + h @ wu2[...] + bu2[...]


def _tc5(s_pad, cid_pad, inc0, inc1, wu1a, wu1b, wu1c, bu1, wu2, bu2):
    g = N_PAD // BN1
    row = lambda i: (i, 0)
    full = lambda i: (0, 0)
    wspec = lambda a: pl.BlockSpec(a.shape, full)
    return pl.pallas_call(
        _tc5_body,
        grid=(g,),
        in_specs=[pl.BlockSpec((BN1, D_s), row), pl.BlockSpec((BN1, D_id), row),
                  pl.BlockSpec((BN1, D_s), row), pl.BlockSpec((BN1, D_s), row)]
        + [wspec(a) for a in (wu1a, wu1b, wu1c, bu1, wu2, bu2)],
        out_specs=pl.BlockSpec((BN1, D_s), row),
        out_shape=jax.ShapeDtypeStruct((N_PAD, D_s), jnp.float32),
    )(s_pad, cid_pad, inc0, inc1, wu1a, wu1b, wu1c, bu1, wu2, bu2)


def kernel(s, out_nbrs, E_bias_flat, col_id, W_c1, b_c1, W_c2, b_c2,
           W_q1, b_q1, W_q2, b_q2, W_k1, b_k1, W_k2, b_k2,
           W_u1, b_u1, W_u2, b_u2):
    f32 = jnp.float32
    s2 = s[0].astype(f32)
    pad = N_PAD - N

    s_pad = jnp.pad(s2, ((0, pad), (0, 0)))
    cid_pad = jnp.pad(col_id.astype(f32), ((0, pad), (0, 0)))
    pad_idx = (jnp.arange(pad * K, dtype=jnp.int32) % N).reshape(pad, K)
    nbr_pad = jnp.concatenate([out_nbrs.astype(jnp.int32), pad_idx], axis=0)
    idx_flat = nbr_pad.reshape(IDXROWS, CH)
    eb_pad = jnp.pad(E_bias_flat.astype(f32).reshape(N, K), ((0, pad), (0, 0)))

    r2 = lambda b: b.reshape(1, -1).astype(f32)
    m, q, kvec = _tc1(
        s_pad, cid_pad,
        W_c1[:D_s].astype(f32), W_c1[D_s:].astype(f32), r2(b_c1),
        W_c2.astype(f32), r2(b_c2),
        W_q1[:D_s].astype(f32), W_q1[D_s:].astype(f32), r2(b_q1),
        W_q2.astype(f32), r2(b_q2),
        W_k1.astype(f32), r2(b_k1), W_k2.astype(f32), r2(b_k2))

    knbr = _sc_gather(kvec, idx_flat)
    msgs = _tc3(q, knbr, eb_pad, m)

    zer = jnp.zeros((CH, D_s), f32)
    parts = _sc_scatter(msgs, idx_flat, zer)

    s_new = _tc5(s_pad, cid_pad, parts[:N_PAD], parts[N_PAD:],
                 W_u1[:D_s].astype(f32), W_u1[D_s:D_s + D_id].astype(f32),
                 W_u1[D_s + D_id:].astype(f32), r2(b_u1),
                 W_u2.astype(f32), r2(b_u2))
    return s_new[:N][None]
